# baseline (speedup 1.0000x reference)
import functools

import jax
import jax.numpy as jnp
from jax import lax
from jax.experimental import pallas as pl
from jax.experimental.pallas import tpu as pltpu
from jax.experimental.pallas import tpu_sc as plsc

MAXH = 3
D = 128
NC, NS, L = 2, 16, 16
NW = NC * NS
CHUNK = 128
NBUF = 4


def _sc_lookup(idx3d, table, m_total):
    m_per_w = m_total // NW
    n_chunks = m_per_w // CHUNK
    assert n_chunks % NBUF == 0
    mesh = plsc.VectorSubcoreMesh(core_axis_name="c", subcore_axis_name="s")

    @functools.partial(
        pl.kernel,
        out_type=jax.ShapeDtypeStruct((m_total, D), jnp.float32),
        mesh=mesh,
        compiler_params=pltpu.CompilerParams(needs_layout_passes=False),
        scratch_types=(
            [pltpu.VMEM((n_chunks, CHUNK), jnp.int32)]
            + [pltpu.VMEM((CHUNK, D), jnp.float32) for _ in range(NBUF)]
            + [pltpu.SemaphoreType.DMA for _ in range(2 * NBUF)]
        ),
    )
    def k(idx_hbm, table_hbm, out_hbm, idx_v, *bufs):
        rows = bufs[:NBUF]
        sg = bufs[NBUF:2 * NBUF]
        so = bufs[2 * NBUF:]
        wid = lax.axis_index("s") * NC + lax.axis_index("c")
        base_row = wid * m_per_w

        pltpu.sync_copy(idx_hbm.at[wid], idx_v)

        def clamp_body(g, _):
            for j in range(CHUNK // L):
                sl = pl.ds(j * L, L)
                idx_v[g, sl] = jnp.clip(idx_v[g, sl], 0, MAXH)
            return 0

        lax.fori_loop(0, n_chunks, clamp_body, 0, unroll=False)

        def gather(g, b):
            return pltpu.make_async_copy(
                table_hbm.at[idx_v.at[g]], rows[b], sg[b])

        def put(g, b):
            return pltpu.make_async_copy(
                rows[b], out_hbm.at[pl.ds(base_row + g * CHUNK, CHUNK)], so[b])

        for b in range(NBUF):
            gather(b, b).start()

        def outer(o, _):
            for b in range(NBUF):
                g = o * NBUF + b
                gather(g, b).wait()
                put(g, b).start()
                put(g, b).wait()
                g2 = g + NBUF

                @pl.when(g2 < n_chunks)
                def _():
                    gather(g2, b).start()
            return 0

        lax.fori_loop(0, n_chunks // NBUF, outer, 0, unroll=False)

    return k(idx3d, table)


def kernel(hop_distances, hop_embedding):
    b, n = hop_distances.shape
    m_total = b * n
    m_per_w = m_total // NW
    idx3d = hop_distances.astype(jnp.int32).reshape(NW, m_per_w // CHUNK, CHUNK)
    out = _sc_lookup(idx3d, hop_embedding.astype(jnp.float32), m_total)
    return out.reshape(b, n, D)


# speedup vs baseline: 11.4760x; 11.4760x over previous
import functools

import jax
import jax.numpy as jnp
from jax import lax
from jax.experimental import pallas as pl
from jax.experimental.pallas import tpu as pltpu
from jax.experimental.pallas import tpu_sc as plsc

MAXH = 3
D = 128
NC, NS, L = 2, 16, 16
NW = NC * NS
CHUNK = 256
NBUF = 2


def _sc_lookup(idx3d, table_flat, m_total):
    m_per_w = m_total // NW
    n_chunks = m_per_w // CHUNK
    n_outer = n_chunks // NBUF
    mesh = plsc.VectorSubcoreMesh(core_axis_name="c", subcore_axis_name="s")

    @functools.partial(
        pl.kernel,
        out_type=jax.ShapeDtypeStruct((m_total * D,), jnp.float32),
        mesh=mesh,
        compiler_params=pltpu.CompilerParams(needs_layout_passes=False),
        scratch_types=(
            [pltpu.VMEM(((MAXH + 1) * D,), jnp.float32),
             pltpu.VMEM((n_chunks, CHUNK), jnp.int32)]
            + [pltpu.VMEM((CHUNK * D,), jnp.float32) for _ in range(NBUF)]
            + [pltpu.SemaphoreType.DMA for _ in range(NBUF)]
        ),
    )
    def k(idx_hbm, table_hbm, out_hbm, table_v, idx_v, *bufs):
        rows = bufs[:NBUF]
        so = bufs[NBUF:]
        wid = lax.axis_index("s") * NC + lax.axis_index("c")
        base = wid * m_per_w

        pltpu.sync_copy(table_hbm, table_v)
        pltpu.sync_copy(idx_hbm.at[wid], idx_v)

        def clamp_body(g, _):
            for j in range(CHUNK // L):
                sl = pl.ds(j * L, L)
                idx_v[g, sl] = jnp.clip(idx_v[g, sl], 0, MAXH) * D
            return 0

        lax.fori_loop(0, n_chunks, clamp_body, 0, unroll=False)

        lane = lax.iota(jnp.int32, L)

        def compute_chunk(g, b):
            def group(t, _):
                addr0 = idx_v[g, pl.ds(t * L, L)]
                for j in range(L):
                    aj = addr0.at[jnp.full((L,), j, jnp.int32)].get(
                        mode="promise_in_bounds") + lane
                    dst = (t * L + j) * D
                    for c in range(0, D, L):
                        val = plsc.load_gather(table_v, [aj + c])
                        rows[b][pl.ds(dst + c, L)] = val
                return 0

            lax.fori_loop(0, CHUNK // L, group, 0, unroll=False)

        def put(g, b):
            return pltpu.make_async_copy(
                rows[b],
                out_hbm.at[pl.ds((base + g * CHUNK) * D, CHUNK * D)],
                so[b])

        for b in range(NBUF):
            compute_chunk(b, b)
            put(b, b).start()

        def outer(o, _):
            for b in range(NBUF):
                g = o * NBUF + b
                put(g, b).wait()
                compute_chunk(g, b)
                put(g, b).start()
            return 0

        lax.fori_loop(1, n_outer, outer, 0, unroll=False)
        for b in range(NBUF):
            put(0, b).wait()

    return k(idx3d, table_flat)


def kernel(hop_distances, hop_embedding):
    b, n = hop_distances.shape
    m_total = b * n
    m_per_w = m_total // NW
    idx3d = hop_distances.astype(jnp.int32).reshape(NW, m_per_w // CHUNK, CHUNK)
    table_flat = hop_embedding.astype(jnp.float32).reshape(-1)
    out = _sc_lookup(idx3d, table_flat, m_total)
    return out.reshape(b, n, D)


# speedup vs baseline: 20.6493x; 1.7993x over previous
import functools

import jax
import jax.numpy as jnp
from jax import lax
from jax.experimental import pallas as pl
from jax.experimental.pallas import tpu as pltpu
from jax.experimental.pallas import tpu_sc as plsc

MAXH = 3
D = 128
NC, NS, L = 2, 16, 16
NW = NC * NS
CHUNK = 256
NBUF = 2


def _sc_lookup(idx3d, table_flat, m_total):
    m_per_w = m_total // NW
    n_chunks = m_per_w // CHUNK
    n_outer = n_chunks // NBUF
    mesh = plsc.VectorSubcoreMesh(core_axis_name="c", subcore_axis_name="s")

    @functools.partial(
        pl.kernel,
        out_type=jax.ShapeDtypeStruct((m_total * D,), jnp.float32),
        mesh=mesh,
        compiler_params=pltpu.CompilerParams(needs_layout_passes=False),
        scratch_types=(
            [pltpu.VMEM(((MAXH + 1) * D,), jnp.float32),
             pltpu.VMEM((n_chunks, CHUNK), jnp.int32)]
            + [pltpu.VMEM((CHUNK * D,), jnp.float32) for _ in range(NBUF)]
            + [pltpu.SemaphoreType.DMA for _ in range(NBUF)]
        ),
    )
    def k(idx_hbm, table_hbm, out_hbm, table_v, idx_v, *bufs):
        rows = bufs[:NBUF]
        so = bufs[NBUF:]
        wid = lax.axis_index("s") * NC + lax.axis_index("c")
        base = wid * m_per_w

        pltpu.sync_copy(table_hbm, table_v)
        pltpu.sync_copy(idx_hbm.at[wid], idx_v)

        lane = lax.iota(jnp.int32, L)

        def compute_chunk(g, b):
            @plsc.parallel_loop(0, CHUNK // L, unroll=2)
            def group(t):
                vec = idx_v[g, pl.ds(t * L, L)]
                addr0 = jnp.clip(vec, 0, MAXH) * D
                for j in range(L):
                    aj = addr0.at[jnp.full((L,), j, jnp.int32)].get(
                        mode="promise_in_bounds") + lane
                    dst = (t * L + j) * D
                    for c in range(0, D, L):
                        val = plsc.load_gather(table_v, [aj + c])
                        rows[b][pl.ds(dst + c, L)] = val

        def put(g, b):
            return pltpu.make_async_copy(
                rows[b],
                out_hbm.at[pl.ds((base + g * CHUNK) * D, CHUNK * D)],
                so[b])

        for b in range(NBUF):
            compute_chunk(b, b)
            put(b, b).start()

        def outer(o, _):
            for b in range(NBUF):
                g = o * NBUF + b
                put(g, b).wait()
                compute_chunk(g, b)
                put(g, b).start()
            return 0

        lax.fori_loop(1, n_outer, outer, 0, unroll=False)
        for b in range(NBUF):
            put(0, b).wait()

    return k(idx3d, table_flat)


def kernel(hop_distances, hop_embedding):
    b, n = hop_distances.shape
    m_total = b * n
    m_per_w = m_total // NW
    idx3d = hop_distances.astype(jnp.int32).reshape(NW, m_per_w // CHUNK, CHUNK)
    table_flat = hop_embedding.astype(jnp.float32).reshape(-1)
    out = _sc_lookup(idx3d, table_flat, m_total)
    return out.reshape(b, n, D)
